# baseline (device time: 43403 ns/iter reference)
import jax
import jax.numpy as jnp
from jax import lax
from jax.experimental import pallas as pl
from jax.experimental.pallas import tpu as pltpu

C = 8


def kernel(ids, E):
    v_per, d = E.shape
    t = ids.shape[0]
    ch = t // C

    my_z = lax.axis_index("z")
    off = my_z * v_per

    ids_c = ids.reshape(C, ch)
    local = ids_c - off
    inb = (local >= 0) & (local < v_per)
    n_k = inb.sum(axis=1).astype(jnp.int32)
    perm = jnp.argsort(jnp.where(inb, 0, 1), axis=1, stable=True)
    pos = perm.astype(jnp.int32).reshape(t)
    safe = jnp.take_along_axis(
        jnp.clip(local, 0, v_per - 1), perm, axis=1
    ).astype(jnp.int32).reshape(t)
    ids_2d = ids.reshape(t, 1)

    def body(safe_sm, pos_sm, n_sm, ids_v, e_hbm, out_ref,
             gath, partial, comm, gsems, send_sems, recv_sems):
        x = lax.axis_index("x")
        y = lax.axis_index("y")
        z = lax.axis_index("z")
        partner = (x, y, 1 - z)
        voff = z * v_per

        barrier_sem = pltpu.get_barrier_semaphore()
        pl.semaphore_signal(
            barrier_sem, inc=1,
            device_id=partner, device_id_type=pl.DeviceIdType.MESH,
        )

        def issue_chunk(k):
            def issue(i, _):
                pltpu.make_async_copy(
                    e_hbm.at[pl.ds(safe_sm[k * ch + i], 1), :],
                    gath.at[pl.ds(k * ch + pos_sm[k * ch + i], 1), :],
                    gsems.at[k],
                ).start()
                return 0
            lax.fori_loop(0, n_sm[k], issue, 0, unroll=False)

        def wait_chunk(k):
            def waitl(i, _):
                pltpu.make_async_copy(
                    e_hbm.at[pl.ds(0, 1), :],
                    gath.at[pl.ds(0, 1), :],
                    gsems.at[k],
                ).wait()
                return 0
            lax.fori_loop(0, n_sm[k], waitl, 0, unroll=False)

        def chunk_rdma(k):
            return pltpu.make_async_remote_copy(
                src_ref=partial.at[pl.ds(k * ch, ch), :],
                dst_ref=comm.at[pl.ds(k * ch, ch), :],
                send_sem=send_sems.at[k],
                recv_sem=recv_sems.at[k],
                device_id=partner,
                device_id_type=pl.DeviceIdType.MESH,
            )

        issue_chunk(0)
        pl.semaphore_wait(barrier_sem, 1)

        for k in range(C):
            if k + 1 < C:
                issue_chunk(k + 1)
            wait_chunk(k)
            sl = pl.ds(k * ch, ch)
            mask = (ids_v[sl] >= voff) & (ids_v[sl] < voff + v_per)
            partial[sl, :] = jnp.where(mask, gath[sl, :], 0.0).astype(
                jnp.bfloat16
            )
            chunk_rdma(k).start()

        for k in range(C):
            chunk_rdma(k).wait()
            sl = pl.ds(k * ch, ch)
            out_ref[sl, :] = partial[sl, :].astype(jnp.float32) + comm[
                sl, :
            ].astype(jnp.float32)

    return pl.pallas_call(
        body,
        out_shape=jax.ShapeDtypeStruct((t, d), jnp.float32),
        in_specs=[
            pl.BlockSpec(memory_space=pltpu.SMEM),
            pl.BlockSpec(memory_space=pltpu.SMEM),
            pl.BlockSpec(memory_space=pltpu.SMEM),
            pl.BlockSpec(memory_space=pltpu.VMEM),
            pl.BlockSpec(memory_space=pltpu.MemorySpace.HBM),
        ],
        out_specs=pl.BlockSpec(memory_space=pltpu.VMEM),
        scratch_shapes=[
            pltpu.VMEM((t, d), jnp.float32),
            pltpu.VMEM((t, d), jnp.bfloat16),
            pltpu.VMEM((t, d), jnp.bfloat16),
            pltpu.SemaphoreType.DMA((C,)),
            pltpu.SemaphoreType.DMA((C,)),
            pltpu.SemaphoreType.DMA((C,)),
        ],
        compiler_params=pltpu.CompilerParams(collective_id=0),
    )(safe, pos, n_k, ids_2d, E)


# device time: 31527 ns/iter; 1.3767x vs baseline; 1.3767x over previous
import jax
import jax.numpy as jnp
from jax import lax
from jax.experimental import pallas as pl
from jax.experimental.pallas import tpu as pltpu

NC = 4


def kernel(ids, E):
    v_per, d = E.shape
    t = ids.shape[0]
    blk = t // 4
    chsz = blk // NC

    my_z = lax.axis_index("z")
    off = my_z * v_per
    local = ids - off
    inb = (local >= 0) & (local < v_per)
    inb_i = inb.astype(jnp.int32)
    safe = jnp.clip(local, 0, v_per - 1).astype(jnp.int32)
    n_qc = inb_i.reshape(4 * NC, chsz).sum(-1)
    ids_2d = ids.reshape(t, 1)

    def body(safe_sm, inb_sm, n_sm, ids_v, e_hbm, out_ref,
             gath, partial, commz, myblk, commx, commy,
             gsems, z_s, z_r, x_s, x_r, y0_s, y0_r, y1_s, y1_r):
        x = lax.axis_index("x")
        y = lax.axis_index("y")
        z = lax.axis_index("z")
        q = 2 * x + y
        qx = 2 * (1 - x) + y
        qy = 2 * x + (1 - y)
        qxy = 2 * (1 - x) + (1 - y)
        pz = (x, y, 1 - z)
        px = (1 - x, y, z)
        py = (x, 1 - y, z)
        base = q * blk
        voff = z * v_per

        barrier_sem = pltpu.get_barrier_semaphore()
        for nbr in (pz, px, py):
            pl.semaphore_signal(
                barrier_sem, inc=1,
                device_id=nbr, device_id_type=pl.DeviceIdType.MESH,
            )

        for c in range(NC):
            def issue(i, _, c=c):
                tok = base + c * chsz + i

                @pl.when(inb_sm[tok] != 0)
                def _():
                    pltpu.make_async_copy(
                        e_hbm.at[pl.ds(safe_sm[tok], 1), :],
                        gath.at[pl.ds(c * chsz + i, 1), :],
                        gsems.at[c],
                    ).start()
                return 0
            lax.fori_loop(0, chsz, issue, 0, unroll=False)

        pl.semaphore_wait(barrier_sem, 3)

        def row_wait(i, _, sem=None):
            pltpu.make_async_copy(
                e_hbm.at[pl.ds(0, 1), :], gath.at[pl.ds(0, 1), :], sem
            ).wait()
            return 0

        def zrdma(c):
            return pltpu.make_async_remote_copy(
                src_ref=partial.at[pl.ds(c * chsz, chsz), :],
                dst_ref=commz.at[pl.ds(c * chsz, chsz), :],
                send_sem=z_s.at[c], recv_sem=z_r.at[c],
                device_id=pz, device_id_type=pl.DeviceIdType.MESH,
            )

        def xrdma(c):
            return pltpu.make_async_remote_copy(
                src_ref=myblk.at[pl.ds(c * chsz, chsz), :],
                dst_ref=commx.at[pl.ds(c * chsz, chsz), :],
                send_sem=x_s.at[c], recv_sem=x_r.at[c],
                device_id=px, device_id_type=pl.DeviceIdType.MESH,
            )

        def y0rdma(c):
            return pltpu.make_async_remote_copy(
                src_ref=myblk.at[pl.ds(c * chsz, chsz), :],
                dst_ref=commy.at[pl.ds(c * chsz, chsz), :],
                send_sem=y0_s.at[c], recv_sem=y0_r.at[c],
                device_id=py, device_id_type=pl.DeviceIdType.MESH,
            )

        def y1rdma(c):
            return pltpu.make_async_remote_copy(
                src_ref=commx.at[pl.ds(c * chsz, chsz), :],
                dst_ref=commy.at[pl.ds(blk + c * chsz, chsz), :],
                send_sem=y1_s.at[c], recv_sem=y1_r.at[c],
                device_id=py, device_id_type=pl.DeviceIdType.MESH,
            )

        for c in range(NC):
            lax.fori_loop(
                0, n_sm[q * NC + c],
                lambda i, _: row_wait(i, _, sem=gsems.at[c]), 0,
            )
            sl = pl.ds(c * chsz, chsz)
            gsl = pl.ds(base + c * chsz, chsz)
            mask = (ids_v[gsl] >= voff) & (ids_v[gsl] < voff + v_per)
            partial[sl, :] = jnp.where(mask, gath[sl, :], 0.0).astype(
                jnp.bfloat16
            )
            zrdma(c).start()

        for c in range(NC):
            zrdma(c).wait()
            sl = pl.ds(c * chsz, chsz)
            myblk[sl, :] = partial[sl, :] + commz[sl, :]
            out_ref[pl.ds(q * blk + c * chsz, chsz), :] = myblk[
                sl, :
            ].astype(jnp.float32)
            xrdma(c).start()
            y0rdma(c).start()

        for c in range(NC):
            xrdma(c).wait()
            sl = pl.ds(c * chsz, chsz)
            out_ref[pl.ds(qx * blk + c * chsz, chsz), :] = commx[
                sl, :
            ].astype(jnp.float32)
            y1rdma(c).start()

        for c in range(NC):
            y0rdma(c).wait()
            out_ref[pl.ds(qy * blk + c * chsz, chsz), :] = commy[
                pl.ds(c * chsz, chsz), :
            ].astype(jnp.float32)
        for c in range(NC):
            y1rdma(c).wait()
            out_ref[pl.ds(qxy * blk + c * chsz, chsz), :] = commy[
                pl.ds(blk + c * chsz, chsz), :
            ].astype(jnp.float32)

    return pl.pallas_call(
        body,
        out_shape=jax.ShapeDtypeStruct((t, d), jnp.float32),
        in_specs=[
            pl.BlockSpec(memory_space=pltpu.SMEM),
            pl.BlockSpec(memory_space=pltpu.SMEM),
            pl.BlockSpec(memory_space=pltpu.SMEM),
            pl.BlockSpec(memory_space=pltpu.VMEM),
            pl.BlockSpec(memory_space=pltpu.MemorySpace.HBM),
        ],
        out_specs=pl.BlockSpec(memory_space=pltpu.VMEM),
        scratch_shapes=[
            pltpu.VMEM((blk, d), jnp.float32),
            pltpu.VMEM((blk, d), jnp.bfloat16),
            pltpu.VMEM((blk, d), jnp.bfloat16),
            pltpu.VMEM((blk, d), jnp.bfloat16),
            pltpu.VMEM((blk, d), jnp.bfloat16),
            pltpu.VMEM((2 * blk, d), jnp.bfloat16),
            pltpu.SemaphoreType.DMA((NC,)),
            pltpu.SemaphoreType.DMA((NC,)),
            pltpu.SemaphoreType.DMA((NC,)),
            pltpu.SemaphoreType.DMA((NC,)),
            pltpu.SemaphoreType.DMA((NC,)),
            pltpu.SemaphoreType.DMA((NC,)),
            pltpu.SemaphoreType.DMA((NC,)),
            pltpu.SemaphoreType.DMA((NC,)),
            pltpu.SemaphoreType.DMA((NC,)),
        ],
        compiler_params=pltpu.CompilerParams(collective_id=0),
    )(safe, inb_i, n_qc, ids_2d, E)


# device time: 28414 ns/iter; 1.5275x vs baseline; 1.1096x over previous
import jax
import jax.numpy as jnp
from jax import lax
from jax.experimental import pallas as pl
from jax.experimental.pallas import tpu as pltpu

NC = 8
UNROLL = 8


def kernel(ids, E):
    v_per, d = E.shape
    t = ids.shape[0]
    blk = t // 4
    chsz = blk // NC

    my_z = lax.axis_index("z")
    off = my_z * v_per
    safe = jnp.clip(ids - off, 0, v_per - 1).astype(jnp.int32)
    ids_2d = ids.reshape(t, 1)

    def body(safe_sm, ids_v, e_hbm, out_ref,
             gath, partial, commz, myblk, commx, commy,
             gsems, z_s, z_r, x_s, x_r, y0_s, y0_r, y1_s, y1_r):
        x = lax.axis_index("x")
        y = lax.axis_index("y")
        z = lax.axis_index("z")
        q = 2 * x + y
        qx = 2 * (1 - x) + y
        qy = 2 * x + (1 - y)
        qxy = 2 * (1 - x) + (1 - y)
        pz = (x, y, 1 - z)
        px = (1 - x, y, z)
        py = (x, 1 - y, z)
        base = q * blk
        voff = z * v_per

        barrier_sem = pltpu.get_barrier_semaphore()
        for nbr in (pz, px, py):
            pl.semaphore_signal(
                barrier_sem, inc=1,
                device_id=nbr, device_id_type=pl.DeviceIdType.MESH,
            )

        def issue_gather(c):
            def issue(i, _):
                tok0 = base + c * chsz + i * UNROLL
                for u in range(UNROLL):
                    pltpu.make_async_copy(
                        e_hbm.at[pl.ds(safe_sm[tok0 + u], 1), :],
                        gath.at[pl.ds(c * chsz + i * UNROLL + u, 1), :],
                        gsems.at[c],
                    ).start()
                return 0
            lax.fori_loop(0, chsz // UNROLL, issue, 0, unroll=False)

        def gwait(c):
            pltpu.make_async_copy(
                e_hbm.at[pl.ds(0, chsz), :],
                gath.at[pl.ds(c * chsz, chsz), :],
                gsems.at[c],
            ).wait()

        def mk_rdma(src, soff, dst, doff, c, ssem, rsem, dev):
            return pltpu.make_async_remote_copy(
                src_ref=src.at[pl.ds(soff + c * chsz, chsz), :],
                dst_ref=dst.at[pl.ds(doff + c * chsz, chsz), :],
                send_sem=ssem.at[c], recv_sem=rsem.at[c],
                device_id=dev, device_id_type=pl.DeviceIdType.MESH,
            )

        def zrdma(c):
            return mk_rdma(partial, 0, commz, 0, c, z_s, z_r, pz)

        def xrdma(c):
            return mk_rdma(myblk, 0, commx, 0, c, x_s, x_r, px)

        def y0rdma(c):
            return mk_rdma(myblk, 0, commy, 0, c, y0_s, y0_r, py)

        def y1rdma(c):
            return mk_rdma(commx, 0, commy, blk, c, y1_s, y1_r, py)

        def s1(c):
            gwait(c)
            sl = pl.ds(c * chsz, chsz)
            gsl = pl.ds(base + c * chsz, chsz)
            mask = (ids_v[gsl] >= voff) & (ids_v[gsl] < voff + v_per)
            partial[sl, :] = jnp.where(mask, gath[sl, :], 0.0).astype(
                jnp.bfloat16
            )
            zrdma(c).start()

        def s2(c):
            zrdma(c).wait()
            sl = pl.ds(c * chsz, chsz)
            myblk[sl, :] = partial[sl, :] + commz[sl, :]
            xrdma(c).start()
            y0rdma(c).start()
            out_ref[pl.ds(q * blk + c * chsz, chsz), :] = myblk[
                sl, :
            ].astype(jnp.float32)

        def s3(c):
            xrdma(c).wait()
            y1rdma(c).start()
            out_ref[pl.ds(qx * blk + c * chsz, chsz), :] = commx[
                pl.ds(c * chsz, chsz), :
            ].astype(jnp.float32)

        issue_gather(0)
        pl.semaphore_wait(barrier_sem, 3)

        for c in range(NC):
            if c + 1 < NC:
                issue_gather(c + 1)
            s1(c)
            if c >= 1:
                s2(c - 1)
            if c >= 2:
                s3(c - 2)
        s2(NC - 1)
        s3(NC - 2)
        s3(NC - 1)

        for c in range(NC):
            y0rdma(c).wait()
            out_ref[pl.ds(qy * blk + c * chsz, chsz), :] = commy[
                pl.ds(c * chsz, chsz), :
            ].astype(jnp.float32)
        for c in range(NC):
            y1rdma(c).wait()
            out_ref[pl.ds(qxy * blk + c * chsz, chsz), :] = commy[
                pl.ds(blk + c * chsz, chsz), :
            ].astype(jnp.float32)

    return pl.pallas_call(
        body,
        out_shape=jax.ShapeDtypeStruct((t, d), jnp.float32),
        in_specs=[
            pl.BlockSpec(memory_space=pltpu.SMEM),
            pl.BlockSpec(memory_space=pltpu.VMEM),
            pl.BlockSpec(memory_space=pltpu.MemorySpace.HBM),
        ],
        out_specs=pl.BlockSpec(memory_space=pltpu.VMEM),
        scratch_shapes=[
            pltpu.VMEM((blk, d), jnp.float32),
            pltpu.VMEM((blk, d), jnp.bfloat16),
            pltpu.VMEM((blk, d), jnp.bfloat16),
            pltpu.VMEM((blk, d), jnp.bfloat16),
            pltpu.VMEM((blk, d), jnp.bfloat16),
            pltpu.VMEM((2 * blk, d), jnp.bfloat16),
            pltpu.SemaphoreType.DMA((NC,)),
            pltpu.SemaphoreType.DMA((NC,)),
            pltpu.SemaphoreType.DMA((NC,)),
            pltpu.SemaphoreType.DMA((NC,)),
            pltpu.SemaphoreType.DMA((NC,)),
            pltpu.SemaphoreType.DMA((NC,)),
            pltpu.SemaphoreType.DMA((NC,)),
            pltpu.SemaphoreType.DMA((NC,)),
            pltpu.SemaphoreType.DMA((NC,)),
        ],
        compiler_params=pltpu.CompilerParams(collective_id=0),
    )(safe, ids_2d, E)


# device time: 27672 ns/iter; 1.5685x vs baseline; 1.0268x over previous
import jax
import jax.numpy as jnp
from jax import lax
from jax.experimental import pallas as pl
from jax.experimental.pallas import tpu as pltpu

NC = 8
UNROLL = 8


def kernel(ids, E):
    v_per, d = E.shape
    t = ids.shape[0]
    blk = t // 4
    chsz = blk // NC

    my_z = lax.axis_index("z")
    off = my_z * v_per
    safe = jnp.clip(ids - off, 0, v_per - 1).astype(jnp.int32)
    ids_2d = ids.reshape(t, 1)

    def body(safe_sm, ids_v, e_hbm, out_ref,
             gath, partial, commz,
             gsems, z_s, z_r, x_s, x_r, y0_s, y0_r, y1_s, y1_r):
        x = lax.axis_index("x")
        y = lax.axis_index("y")
        z = lax.axis_index("z")
        q = 2 * x + y
        qx = 2 * (1 - x) + y
        pz = (x, y, 1 - z)
        px = (1 - x, y, z)
        py = (x, 1 - y, z)
        base = q * blk
        voff = z * v_per

        barrier_sem = pltpu.get_barrier_semaphore()
        for nbr in (pz, px, py):
            pl.semaphore_signal(
                barrier_sem, inc=1,
                device_id=nbr, device_id_type=pl.DeviceIdType.MESH,
            )

        def issue_gather(c):
            def issue(i, _):
                tok0 = base + c * chsz + i * UNROLL
                for u in range(UNROLL):
                    pltpu.make_async_copy(
                        e_hbm.at[pl.ds(safe_sm[tok0 + u], 1), :],
                        gath.at[pl.ds(c * chsz + i * UNROLL + u, 1), :],
                        gsems.at[c],
                    ).start()
                return 0
            lax.fori_loop(0, chsz // UNROLL, issue, 0, unroll=False)

        def gwait(c):
            pltpu.make_async_copy(
                e_hbm.at[pl.ds(0, chsz), :],
                gath.at[pl.ds(c * chsz, chsz), :],
                gsems.at[c],
            ).wait()

        def mk_rdma(src, soff, dst, doff, c, ssem, rsem, dev):
            return pltpu.make_async_remote_copy(
                src_ref=src.at[pl.ds(soff + c * chsz, chsz), :],
                dst_ref=dst.at[pl.ds(doff + c * chsz, chsz), :],
                send_sem=ssem.at[c], recv_sem=rsem.at[c],
                device_id=dev, device_id_type=pl.DeviceIdType.MESH,
            )

        def zrdma(c):
            return mk_rdma(partial, 0, commz, 0, c, z_s, z_r, pz)

        def xrdma(c):
            return mk_rdma(out_ref, base, out_ref, base, c, x_s, x_r, px)

        def y0rdma(c):
            return mk_rdma(out_ref, base, out_ref, base, c, y0_s, y0_r, py)

        def y1rdma(c):
            return mk_rdma(
                out_ref, qx * blk, out_ref, qx * blk, c, y1_s, y1_r, py
            )

        def s1(c):
            gwait(c)
            sl = pl.ds(c * chsz, chsz)
            gsl = pl.ds(base + c * chsz, chsz)
            mask = (ids_v[gsl] >= voff) & (ids_v[gsl] < voff + v_per)
            partial[sl, :] = jnp.where(mask, gath[sl, :], 0.0).astype(
                jnp.bfloat16
            )
            zrdma(c).start()

        def s2(c):
            zrdma(c).wait()
            sl = pl.ds(c * chsz, chsz)
            out_ref[pl.ds(base + c * chsz, chsz), :] = (
                partial[sl, :] + commz[sl, :]
            )
            xrdma(c).start()
            y0rdma(c).start()

        def s3(c):
            xrdma(c).wait()
            y1rdma(c).start()

        issue_gather(0)
        pl.semaphore_wait(barrier_sem, 3)

        for c in range(NC):
            if c + 1 < NC:
                issue_gather(c + 1)
            s1(c)
            if c >= 1:
                s2(c - 1)
            if c >= 2:
                s3(c - 2)
        s2(NC - 1)
        s3(NC - 2)
        s3(NC - 1)

        for c in range(NC):
            y0rdma(c).wait()
        for c in range(NC):
            y1rdma(c).wait()

    return pl.pallas_call(
        body,
        out_shape=jax.ShapeDtypeStruct((t, d), jnp.bfloat16),
        in_specs=[
            pl.BlockSpec(memory_space=pltpu.SMEM),
            pl.BlockSpec(memory_space=pltpu.VMEM),
            pl.BlockSpec(memory_space=pltpu.MemorySpace.HBM),
        ],
        out_specs=pl.BlockSpec(memory_space=pltpu.VMEM),
        scratch_shapes=[
            pltpu.VMEM((blk, d), jnp.float32),
            pltpu.VMEM((blk, d), jnp.bfloat16),
            pltpu.VMEM((blk, d), jnp.bfloat16),
            pltpu.SemaphoreType.DMA((NC,)),
            pltpu.SemaphoreType.DMA((NC,)),
            pltpu.SemaphoreType.DMA((NC,)),
            pltpu.SemaphoreType.DMA((NC,)),
            pltpu.SemaphoreType.DMA((NC,)),
            pltpu.SemaphoreType.DMA((NC,)),
            pltpu.SemaphoreType.DMA((NC,)),
            pltpu.SemaphoreType.DMA((NC,)),
            pltpu.SemaphoreType.DMA((NC,)),
        ],
        compiler_params=pltpu.CompilerParams(collective_id=0),
    )(safe, ids_2d, E)
